# baseline (speedup 1.0000x reference)
import jax
import jax.numpy as jnp
from jax import lax
from jax.experimental import pallas as pl
from jax.experimental.pallas import tpu as pltpu
from jax.experimental.pallas import tpu_sc as plsc

_N = 50000
_E = 800000
_H = 32
_L = 3
_GW = 128
_NG = _E // _GW
_NC = 2
_NS = 16
_NW = _NC * _NS
_TG = 200
_NGPAD = _TG * _NW
_NP = 51200
_BR = 3200
_GRID = _NP // _BR

_BKT = 1568
_CAPT = 1024
_RTOT = _NW * _NW * _CAPT
_RPTC = _NW * _CAPT
_CH = 1024
_TRASH = _BKT
_CROW = 1600


def _mesh():
    return plsc.VectorSubcoreMesh(core_axis_name="c", subcore_axis_name="s")


def _partition(idx):
    def body(idx_ref, recs_ref, ibuf, stage_v, stage_i, tbuf, cnt, tmp16,
             sem):
        cid = lax.axis_index("c")
        sid = lax.axis_index("s")
        wid = cid * _NS + sid
        iota = lax.broadcasted_iota(jnp.int32, (16,), 0)
        zeros16 = jnp.zeros((16,), jnp.int32)
        trash16 = jnp.full((16,), _TRASH << 16, jnp.int32)

        def fill(i, c):
            tbuf[pl.ds(i * 16, 16)] = trash16
            return c
        lax.fori_loop(0, _CAPT // 16, fill, 0)
        cnt[pl.ds(0, 16)] = zeros16
        cnt[pl.ds(16, 16)] = zeros16
        cps = []
        for b in range(_NW):
            cps.append(pltpu.async_copy(
                tbuf, recs_ref.at[pl.ds((b * _NW + wid) * _CAPT, _CAPT)],
                sem))
        for cp in cps:
            cp.wait()

        def chunk(ci, c):
            base = wid * _TG + ci * 8
            pltpu.sync_copy(idx_ref.at[pl.ds(base, 8)], ibuf)
            for g in range(64):
                j, r = g // 8, g % 8
                srcv = ibuf[j, 0, pl.ds(r * 16, 16)]
                dstv = ibuf[j, 1, pl.ds(r * 16, 16)]
                b = lax.div(dstv, _BKT)
                dloc = dstv - b * _BKT
                rec = jnp.bitwise_or(srcv, jnp.left_shift(dloc, 16))
                bs, rs = plsc.sort_key_val(b, rec)
                tmp16[...] = bs
                prev = plsc.load_gather(tmp16, [jnp.maximum(iota - 1, 0)])
                nxt = plsc.load_gather(tmp16, [jnp.minimum(iota + 1, 15)])
                newrun = (iota == 0) | (bs != prev)
                lastrun = (iota == 15) | (bs != nxt)
                f = plsc.cummax(jnp.where(newrun, iota, 0))
                rank = iota - f
                basec = plsc.load_gather(cnt, [bs])
                pos = basec + rank
                plsc.store_scatter(cnt, [bs], pos + 1, mask=lastrun)
                pos = jnp.minimum(pos, _CAPT - 1)
                gpos = (bs * _NW + wid) * _CAPT + pos
                stage_v[j, pl.ds(r * 16, 16)] = rs
                stage_i[j, pl.ds(r * 16, 16)] = gpos
            scps = []
            for j in range(8):
                scps.append(pltpu.async_copy(
                    stage_v.at[j], recs_ref.at[stage_i.at[j]], sem))
            for cp in scps:
                cp.wait()
            return c

        lax.fori_loop(0, _TG // 8, chunk, 0)

    f = pl.kernel(
        body,
        out_type=jax.ShapeDtypeStruct((_RTOT,), jnp.int32),
        mesh=_mesh(),
        scratch_types=(
            pltpu.VMEM((8, 2, _GW), jnp.int32),
            pltpu.VMEM((8, _GW), jnp.int32),
            pltpu.VMEM((8, _GW), jnp.int32),
            pltpu.VMEM((_CAPT,), jnp.int32),
            pltpu.VMEM((_NW,), jnp.int32),
            pltpu.VMEM((16,), jnp.int32),
            pltpu.SemaphoreType.DMA,
        ),
        compiler_params=pltpu.CompilerParams(use_tc_tiling_on_sc=False,
                                             needs_layout_passes=False))
    return f(idx)


def _bucket_pass(h, recs, zacc, with_counts):
    nacc = 2 * _CROW if with_counts else _CROW
    out_type = [jax.ShapeDtypeStruct((_NP, _H), jnp.float32)]
    if with_counts:
        out_type.append(jax.ShapeDtypeStruct((_NP, _H), jnp.float32))

    def body(*refs):
        if with_counts:
            (h_ref, recs_ref, z_ref, out_ref, cnt_ref,
             rbuf, sidx, rows, acc, s0, s1, s2, s3) = refs
        else:
            (h_ref, recs_ref, z_ref, out_ref,
             rbuf, sidx, rows, acc, s0, s1, s2, s3) = refs
        sems = (s0, s1, s2, s3)
        cid = lax.axis_index("c")
        sid = lax.axis_index("s")
        wid = cid * _NS + sid
        iota = lax.broadcasted_iota(jnp.int32, (16,), 0)
        one0 = jnp.where(iota == 0, 1.0, 0.0).astype(jnp.float32)

        pltpu.sync_copy(z_ref.at[pl.ds(0, nacc)], acc)
        base = wid * _RPTC

        def chunk(ci, c):
            pltpu.sync_copy(recs_ref.at[pl.ds(base + ci * _CH, _CH)], rbuf)
            for g in range(64):
                v = rbuf[pl.ds(g * 16, 16)]
                sidx[g // 8, pl.ds((g % 8) * 16, 16)] = \
                    jnp.bitwise_and(v, 0xFFFF)
            cps = [None] * 4
            cps[0] = pltpu.async_copy(h_ref.at[sidx.at[0]], rows.at[0],
                                      sems[0])
            for j in range(8):
                for a in range(1, 4):
                    if j + a < 8 and (j == 0 or a == 3):
                        nb = (j + a) % 4
                        cps[nb] = pltpu.async_copy(
                            h_ref.at[sidx.at[j + a]], rows.at[nb], sems[nb])
                cps[j % 4].wait()
                for r in range(8):
                    v = rbuf[pl.ds((j * 8 + r) * 16, 16)]
                    dvv = lax.shift_right_logical(v, 16)
                    for e2 in range(16):
                        dv = dvv[e2]
                        e = r * 16 + e2
                        plsc.addupdate(acc.at[dv, pl.ds(0, 16)],
                                       rows[j % 4, e, pl.ds(0, 16)])
                        plsc.addupdate(acc.at[dv, pl.ds(16, 16)],
                                       rows[j % 4, e, pl.ds(16, 16)])
                        if with_counts:
                            plsc.addupdate(
                                acc.at[dv + _CROW, pl.ds(0, 16)], one0)
            return c

        lax.fori_loop(0, _RPTC // _CH, chunk, 0)
        pltpu.sync_copy(acc.at[pl.ds(0, _BKT)],
                        out_ref.at[pl.ds(wid * _BKT, _BKT)])
        if with_counts:
            pltpu.sync_copy(acc.at[pl.ds(_CROW, _BKT)],
                            cnt_ref.at[pl.ds(wid * _BKT, _BKT)])

        @pl.when(wid == _NW - 1)
        def _tail():
            pad = _NP - _NW * _BKT
            pltpu.sync_copy(z_ref.at[pl.ds(0, pad)],
                            out_ref.at[pl.ds(_NW * _BKT, pad)])
            if with_counts:
                pltpu.sync_copy(z_ref.at[pl.ds(0, pad)],
                                cnt_ref.at[pl.ds(_NW * _BKT, pad)])

    f = pl.kernel(
        body,
        out_type=tuple(out_type),
        mesh=_mesh(),
        scratch_types=(
            pltpu.VMEM((_CH,), jnp.int32),
            pltpu.VMEM((8, _GW), jnp.int32),
            pltpu.VMEM((4, _GW, _H), jnp.float32),
            pltpu.VMEM((nacc, _H), jnp.float32),
            pltpu.SemaphoreType.DMA,
            pltpu.SemaphoreType.DMA,
            pltpu.SemaphoreType.DMA,
            pltpu.SemaphoreType.DMA,
        ),
        compiler_params=pltpu.CompilerParams(use_tc_tiling_on_sc=False,
                                             needs_layout_passes=False))
    return f(h, recs, zacc)


def _tc_init(det, errf, wdet, bdet_, werr, berr_):
    def body(d_ref, e_ref, wd_ref, bd_ref, we_ref, be_ref, hd_ref, he_ref):
        hd_ref[...] = jnp.maximum(d_ref[...] * wd_ref[...] + bd_ref[...], 0.0)
        he_ref[...] = jnp.maximum(e_ref[...] * we_ref[...] + be_ref[...], 0.0)

    row = pl.BlockSpec((_BR, 1), lambda i: (i, 0))
    par = pl.BlockSpec((1, _H), lambda i: (0, 0))
    f = pl.pallas_call(
        body, grid=(_GRID,),
        in_specs=[row, row, par, par, par, par],
        out_specs=[pl.BlockSpec((_BR, _H), lambda i: (i, 0))] * 2,
        out_shape=[jax.ShapeDtypeStruct((_NP, _H), jnp.float32)] * 2,
    )
    return f(det, errf, wdet, bdet_, werr, berr_)


def _tc_update(h, s, c0, ew, wagg, wself, bias, g, b, weighted):
    def body(*refs):
        if weighted:
            (h_ref, s_ref, c_ref, ew_ref,
             wa_ref, ws_ref, bi_ref, g_ref, b_ref, o_ref) = refs
        else:
            (h_ref, s_ref, c_ref,
             wa_ref, ws_ref, bi_ref, g_ref, b_ref, o_ref) = refs
        agg = jnp.dot(s_ref[...], wa_ref[...],
                      preferred_element_type=jnp.float32)
        cnt = jnp.maximum(c_ref[...], 1.0)
        if weighted:
            scale = (1.0 / (1.0 + jnp.exp(-ew_ref[...]))) / cnt
        else:
            scale = 1.0 / cnt
        hcur = h_ref[...]
        pre = (jnp.dot(hcur, ws_ref[...], preferred_element_type=jnp.float32)
               + agg * scale + bi_ref[...])
        t = hcur + jnp.maximum(pre, 0.0)
        mu = jnp.mean(t, axis=1, keepdims=True)
        d = t - mu
        var = jnp.mean(d * d, axis=1, keepdims=True)
        o_ref[...] = d * lax.rsqrt(var + 1e-5) * g_ref[...] + b_ref[...]

    blk = pl.BlockSpec((_BR, _H), lambda i: (i, 0))
    col = pl.BlockSpec((_BR, 1), lambda i: (i, 0))
    wsp = pl.BlockSpec((_H, _H), lambda i: (0, 0))
    par = pl.BlockSpec((1, _H), lambda i: (0, 0))
    in_specs = [blk, blk, col]
    args = [h, s, c0]
    if weighted:
        in_specs.append(col)
        args.append(ew)
    in_specs += [wsp, wsp, par, par, par]
    args += [wagg, wself, bias, g, b]
    f = pl.pallas_call(
        body, grid=(_GRID,), in_specs=in_specs,
        out_specs=pl.BlockSpec((_BR, _H), lambda i: (i, 0)),
        out_shape=jax.ShapeDtypeStruct((_NP, _H), jnp.float32),
    )
    return f(*args)


def _tc_pool(he, maskf):
    def body(h_ref, m_ref, o_ref):
        i = pl.program_id(0)

        @pl.when(i == 0)
        def _init():
            o_ref[...] = jnp.zeros((8, _H), jnp.float32)
            o_ref[2:4, :] = jnp.full((2, _H), -jnp.inf, jnp.float32)

        h = h_ref[...]
        m = m_ref[...]
        rowid = (i * _BR
                 + lax.broadcasted_iota(jnp.int32, (_BR, 1), 0))
        valid = rowid < _N
        o_ref[0:1, :] += jnp.sum(jnp.where(m > 0.0, h, 0.0), axis=0,
                                 keepdims=True)
        o_ref[1:2, :] += jnp.sum(jnp.where(valid, h, 0.0), axis=0,
                                 keepdims=True)
        o_ref[2:3, :] = jnp.maximum(
            o_ref[2:3, :],
            jnp.max(jnp.where(m > 0.0, h, -jnp.inf), axis=0, keepdims=True))
        o_ref[3:4, :] = jnp.maximum(
            o_ref[3:4, :],
            jnp.max(jnp.where(valid, h, -jnp.inf), axis=0, keepdims=True))
        o_ref[4:5, :] += jnp.sum(m) * jnp.ones((1, _H), jnp.float32)

    f = pl.pallas_call(
        body, grid=(_GRID,),
        in_specs=[pl.BlockSpec((_BR, _H), lambda i: (i, 0)),
                  pl.BlockSpec((_BR, 1), lambda i: (i, 0))],
        out_specs=pl.BlockSpec((8, _H), lambda i: (0, 0)),
        out_shape=jax.ShapeDtypeStruct((8, _H), jnp.float32),
    )
    return f(he, maskf)


def _tc_head(stats, wh1, bh1_, wh2, bh2_):
    def body(s_ref, w1_ref, b1_ref, w2_ref, b2_ref, o_ref):
        s = s_ref[...]
        cnt = s[4:5, 0:1]
        use = cnt > 0.0
        mean_m = s[0:1, :] / jnp.maximum(cnt, 1.0)
        mean_p = s[1:2, :] * (1.0 / _N)
        mean_e = jnp.where(use, mean_m, mean_p)
        max_e = jnp.where(use, s[2:3, :], s[3:4, :])
        emb = jnp.concatenate([mean_e, max_e], axis=1)
        hmid = jnp.maximum(
            jnp.dot(emb, w1_ref[...], preferred_element_type=jnp.float32)
            + b1_ref[...], 0.0)
        o_ref[...] = (jnp.dot(hmid, w2_ref[...],
                              preferred_element_type=jnp.float32)
                      + b2_ref[...])

    f = pl.pallas_call(
        body, out_shape=jax.ShapeDtypeStruct((1, 1), jnp.float32))
    return f(stats, wh1, bh1_, wh2, bh2_)


def kernel(det_features, err_features, edge_index_d2e, edge_index_e2d,
           error_weights, observable_mask, Wdet, bdet, Werr, berr, Wd2e,
           We_self, be, ln_e_g, ln_e_b, We2d, Wd_self, bd, ln_d_g, ln_d_b,
           Wh1, bh1, Wh2, bh2):
    f32 = jnp.float32

    def _padrows(v):
        return jnp.pad(v, ((0, _NP - _N), (0, 0)))

    det = _padrows(det_features.reshape(_N, 1).astype(f32))
    errf = _padrows(err_features.reshape(_N, 1).astype(f32))

    def _prep_idx(ei):
        src = ei[0].reshape(_NG, _GW)
        dst = ei[1].reshape(_NG, _GW)
        both = jnp.stack([src, dst], axis=1)
        return jnp.pad(both, ((0, _NGPAD - _NG), (0, 0), (0, 0)),
                       constant_values=_N)

    idx_d2e = _prep_idx(edge_index_d2e)
    idx_e2d = _prep_idx(edge_index_e2d)
    zacc = jnp.zeros((2 * _CROW, _H), f32)
    ew2 = _padrows(error_weights.reshape(_N, 1).astype(f32))
    maskf = _padrows(observable_mask.reshape(_N, 1).astype(f32))

    recs_d2e = _partition(idx_d2e)
    recs_e2d = _partition(idx_e2d)
    hD, hE = _tc_init(det, errf, Wdet.reshape(1, _H), bdet.reshape(1, _H),
                      Werr.reshape(1, _H), berr.reshape(1, _H))

    ce = cd = None
    for k in range(_L):
        if k == 0:
            S, ce2 = _bucket_pass(hD, recs_d2e, zacc, True)
            ce = ce2[:, 0:1]
        else:
            (S,) = _bucket_pass(hD, recs_d2e, zacc, False)
        hE = _tc_update(hE, S, ce, ew2, Wd2e[k], We_self[k],
                        be[k].reshape(1, _H), ln_e_g[k].reshape(1, _H),
                        ln_e_b[k].reshape(1, _H), True)
        if k == 0:
            T, cd2 = _bucket_pass(hE, recs_e2d, zacc, True)
            cd = cd2[:, 0:1]
        else:
            (T,) = _bucket_pass(hE, recs_e2d, zacc, False)
        hD = _tc_update(hD, T, cd, None, We2d[k], Wd_self[k],
                        bd[k].reshape(1, _H), ln_d_g[k].reshape(1, _H),
                        ln_d_b[k].reshape(1, _H), False)

    stats = _tc_pool(hE, maskf)
    return _tc_head(stats, Wh1, bh1.reshape(1, _H), Wh2, bh2.reshape(1, 1))


# speedup vs baseline: 6.8650x; 6.8650x over previous
import functools

import jax
import jax.numpy as jnp
from jax import lax
from jax.experimental import pallas as pl
from jax.experimental.pallas import tpu as pltpu
from jax.experimental.pallas import tpu_sc as plsc

_N = 50000
_E = 800000
_H = 32
_L = 3
_GW = 128
_NG = _E // _GW
_NC = 2
_NS = 16
_NW = _NC * _NS
_TG = 200
_NGPAD = _TG * _NW
_CG = 8
_NCH = _TG // _CG
_NP = 51200
_SPT = _NP // _NS
_BR = 3128
_GRID = _NP // _BR


def _edge_pass(h, srci, dsti, z2d, z1d, with_counts):
    mesh = plsc.VectorSubcoreMesh(core_axis_name="c", subcore_axis_name="s")
    out_type = [jax.ShapeDtypeStruct((_NC, _NP, _H), jnp.float32)]
    scratch = [
        pltpu.VMEM((_CG, _GW), jnp.int32),
        pltpu.VMEM((_CG, _GW), jnp.int32),
        pltpu.VMEM((2, _GW, _H), jnp.float32),
        pltpu.SemaphoreType.DMA,
        pltpu.SemaphoreType.DMA,
        pltpu.SemaphoreType.DMA,
        pltpu.SemaphoreType.DMA,
    ]
    if with_counts:
        out_type.append(jax.ShapeDtypeStruct((_NC * _NP,), jnp.float32))
        scratch.append(pltpu.VMEM_SHARED((_NP,), jnp.float32))
        scratch.append(pltpu.VMEM((_GW,), jnp.float32))
    scratch.append(pltpu.VMEM_SHARED((_NP, _H), jnp.float32))

    def body(*refs):
        if with_counts:
            (h_ref, s_ref, d_ref, z2_ref, z1_ref, out_ref, cnt_ref,
             sbuf, dbuf, rows, sem0, sem1, sem2, sem3,
             acc1, ones_v, acc) = refs
        else:
            (h_ref, s_ref, d_ref, z2_ref, z1_ref, out_ref,
             sbuf, dbuf, rows, sem0, sem1, sem2, sem3, acc) = refs
        sems = (sem0, sem1)
        ssems = (sem2, sem3)
        cid = lax.axis_index("c")
        sid = lax.axis_index("s")
        wid = cid * _NS + sid
        pltpu.sync_copy(z2_ref, acc.at[pl.ds(sid * _SPT, _SPT)])
        if with_counts:
            pltpu.sync_copy(z1_ref, acc1.at[pl.ds(sid * _SPT, _SPT)])
            for j in range(_GW // 16):
                ones_v[pl.ds(j * 16, 16)] = jnp.ones((16,), jnp.float32)
        start = wid * _TG
        plsc.subcore_barrier()

        def chunk(ci, c):
            base = start + ci * _CG
            pltpu.sync_copy(s_ref.at[pl.ds(base, _CG)], sbuf)
            pltpu.sync_copy(d_ref.at[pl.ds(base, _CG)], dbuf)
            cps = [None, None]
            scat = [None, None]
            cps[0] = pltpu.async_copy(h_ref.at[sbuf.at[0]], rows.at[0],
                                      sems[0])
            for j in range(_CG):
                if j + 1 < _CG:
                    nb = (j + 1) % 2
                    if scat[nb] is not None:
                        scat[nb].wait()
                    cps[nb] = pltpu.async_copy(h_ref.at[sbuf.at[j + 1]],
                                               rows.at[nb], sems[nb])
                cps[j % 2].wait()
                scat[j % 2] = pltpu.async_copy(
                    rows.at[j % 2], acc.at[dbuf.at[j]], ssems[j % 2],
                    add=True)
                if with_counts:
                    pltpu.sync_copy(ones_v, acc1.at[dbuf.at[j]], add=True)
            for b2 in range(2):
                if scat[b2] is not None:
                    scat[b2].wait()
            return c

        lax.fori_loop(0, _NCH, chunk, 0)
        plsc.subcore_barrier()
        pltpu.sync_copy(acc.at[pl.ds(sid * _SPT, _SPT)],
                        out_ref.at[cid, pl.ds(sid * _SPT, _SPT)])
        if with_counts:
            pltpu.sync_copy(acc1.at[pl.ds(sid * _SPT, _SPT)],
                            cnt_ref.at[pl.ds(cid * _NP + sid * _SPT, _SPT)])

    f = pl.kernel(body, out_type=tuple(out_type), mesh=mesh,
                  scratch_types=tuple(scratch),
                  compiler_params=pltpu.CompilerParams(
                      use_tc_tiling_on_sc=False))
    return f(h, srci, dsti, z2d, z1d)


def _tc_init(det, errf, wdet, bdet_, werr, berr_):
    def body(d_ref, e_ref, wd_ref, bd_ref, we_ref, be_ref, hd_ref, he_ref):
        hd_ref[...] = jnp.maximum(d_ref[...] * wd_ref[...] + bd_ref[...], 0.0)
        he_ref[...] = jnp.maximum(e_ref[...] * we_ref[...] + be_ref[...], 0.0)

    row = pl.BlockSpec((_BR, 1), lambda i: (i, 0))
    par = pl.BlockSpec((1, _H), lambda i: (0, 0))
    f = pl.pallas_call(
        body, grid=(_GRID,),
        in_specs=[row, row, par, par, par, par],
        out_specs=[pl.BlockSpec((_BR, _H), lambda i: (i, 0))] * 2,
        out_shape=[jax.ShapeDtypeStruct((_NP, _H), jnp.float32)] * 2,
    )
    return f(det, errf, wdet, bdet_, werr, berr_)


def _tc_update(h, p0, p1, c0, c1, ew, wagg, wself, bias, g, b, weighted):
    def body(*refs):
        if weighted:
            (h_ref, p0_ref, p1_ref, c0_ref, c1_ref, ew_ref,
             wa_ref, ws_ref, bi_ref, g_ref, b_ref, o_ref) = refs
        else:
            (h_ref, p0_ref, p1_ref, c0_ref, c1_ref,
             wa_ref, ws_ref, bi_ref, g_ref, b_ref, o_ref) = refs
        s = p0_ref[...] + p1_ref[...]
        agg = jnp.dot(s, wa_ref[...], preferred_element_type=jnp.float32)
        cnt = jnp.maximum(c0_ref[...] + c1_ref[...], 1.0)
        if weighted:
            scale = (1.0 / (1.0 + jnp.exp(-ew_ref[...]))) / cnt
        else:
            scale = 1.0 / cnt
        hcur = h_ref[...]
        pre = (jnp.dot(hcur, ws_ref[...], preferred_element_type=jnp.float32)
               + agg * scale + bi_ref[...])
        t = hcur + jnp.maximum(pre, 0.0)
        mu = jnp.mean(t, axis=1, keepdims=True)
        d = t - mu
        var = jnp.mean(d * d, axis=1, keepdims=True)
        o_ref[...] = d * lax.rsqrt(var + 1e-5) * g_ref[...] + b_ref[...]

    blk = pl.BlockSpec((_BR, _H), lambda i: (i, 0))
    col = pl.BlockSpec((_BR, 1), lambda i: (i, 0))
    wsp = pl.BlockSpec((_H, _H), lambda i: (0, 0))
    par = pl.BlockSpec((1, _H), lambda i: (0, 0))
    in_specs = [blk, blk, blk, col, col]
    args = [h, p0, p1, c0, c1]
    if weighted:
        in_specs.append(col)
        args.append(ew)
    in_specs += [wsp, wsp, par, par, par]
    args += [wagg, wself, bias, g, b]
    f = pl.pallas_call(
        body, grid=(_GRID,), in_specs=in_specs,
        out_specs=pl.BlockSpec((_BR, _H), lambda i: (i, 0)),
        out_shape=jax.ShapeDtypeStruct((_NP, _H), jnp.float32),
    )
    return f(*args)


def _tc_pool(he, maskf):
    def body(h_ref, m_ref, o_ref):
        i = pl.program_id(0)

        @pl.when(i == 0)
        def _init():
            o_ref[...] = jnp.zeros((8, _H), jnp.float32)
            o_ref[2:4, :] = jnp.full((2, _H), -jnp.inf, jnp.float32)

        h = h_ref[...]
        m = m_ref[...]
        rowid = (i * _BR
                 + lax.broadcasted_iota(jnp.int32, (_BR, 1), 0))
        valid = rowid < _N
        o_ref[0:1, :] += jnp.sum(h * m, axis=0, keepdims=True)
        o_ref[1:2, :] += jnp.sum(jnp.where(valid, h, 0.0), axis=0,
                                 keepdims=True)
        o_ref[2:3, :] = jnp.maximum(
            o_ref[2:3, :],
            jnp.max(jnp.where(m > 0.0, h, -jnp.inf), axis=0, keepdims=True))
        o_ref[3:4, :] = jnp.maximum(
            o_ref[3:4, :],
            jnp.max(jnp.where(valid, h, -jnp.inf), axis=0, keepdims=True))
        o_ref[4:5, :] += jnp.sum(m) * jnp.ones((1, _H), jnp.float32)

    f = pl.pallas_call(
        body, grid=(_GRID,),
        in_specs=[pl.BlockSpec((_BR, _H), lambda i: (i, 0)),
                  pl.BlockSpec((_BR, 1), lambda i: (i, 0))],
        out_specs=pl.BlockSpec((8, _H), lambda i: (0, 0)),
        out_shape=jax.ShapeDtypeStruct((8, _H), jnp.float32),
    )
    return f(he, maskf)


def _tc_head(stats, wh1, bh1_, wh2, bh2_):
    def body(s_ref, w1_ref, b1_ref, w2_ref, b2_ref, o_ref):
        s = s_ref[...]
        cnt = s[4:5, 0:1]
        use = cnt > 0.0
        mean_m = s[0:1, :] / jnp.maximum(cnt, 1.0)
        mean_p = s[1:2, :] * (1.0 / _N)
        mean_e = jnp.where(use, mean_m, mean_p)
        max_e = jnp.where(use, s[2:3, :], s[3:4, :])
        emb = jnp.concatenate([mean_e, max_e], axis=1)
        hmid = jnp.maximum(
            jnp.dot(emb, w1_ref[...], preferred_element_type=jnp.float32)
            + b1_ref[...], 0.0)
        o_ref[...] = (jnp.dot(hmid, w2_ref[...],
                              preferred_element_type=jnp.float32)
                      + b2_ref[...])

    f = pl.pallas_call(
        body, out_shape=jax.ShapeDtypeStruct((1, 1), jnp.float32))
    return f(stats, wh1, bh1_, wh2, bh2_)


def kernel(det_features, err_features, edge_index_d2e, edge_index_e2d,
           error_weights, observable_mask, Wdet, bdet, Werr, berr, Wd2e,
           We_self, be, ln_e_g, ln_e_b, We2d, Wd_self, bd, ln_d_g, ln_d_b,
           Wh1, bh1, Wh2, bh2):
    f32 = jnp.float32

    def _padrows(v):
        return jnp.pad(v, ((0, _NP - _N), (0, 0)))

    det = _padrows(det_features.reshape(_N, 1).astype(f32))
    errf = _padrows(err_features.reshape(_N, 1).astype(f32))

    def _prep_idx(v, pad):
        v = v.reshape(_NG, _GW)
        return jnp.pad(v, ((0, _NGPAD - _NG), (0, 0)), constant_values=pad)

    d_src = _prep_idx(edge_index_d2e[0], 0)
    e_dst = _prep_idx(edge_index_d2e[1], _N)
    e_src = _prep_idx(edge_index_e2d[0], 0)
    d_dst = _prep_idx(edge_index_e2d[1], _N)
    z2d = jnp.zeros((_SPT, _H), f32)
    z1d = jnp.zeros((_SPT,), f32)
    ew2 = _padrows(error_weights.reshape(_N, 1).astype(f32))
    maskf = _padrows(observable_mask.reshape(_N, 1).astype(f32))

    hD, hE = _tc_init(det, errf, Wdet.reshape(1, _H), bdet.reshape(1, _H),
                      Werr.reshape(1, _H), berr.reshape(1, _H))

    ce0 = ce1 = cd0 = cd1 = None
    for k in range(_L):
        if k == 0:
            S, cep = _edge_pass(hD, d_src, e_dst, z2d, z1d, True)
            ce0 = cep[:_NP].reshape(_NP, 1)
            ce1 = cep[_NP:].reshape(_NP, 1)
        else:
            (S,) = _edge_pass(hD, d_src, e_dst, z2d, z1d, False)
        hE = _tc_update(hE, S[0], S[1], ce0, ce1, ew2, Wd2e[k], We_self[k],
                        be[k].reshape(1, _H), ln_e_g[k].reshape(1, _H),
                        ln_e_b[k].reshape(1, _H), True)
        if k == 0:
            T, cdp = _edge_pass(hE, e_src, d_dst, z2d, z1d, True)
            cd0 = cdp[:_NP].reshape(_NP, 1)
            cd1 = cdp[_NP:].reshape(_NP, 1)
        else:
            (T,) = _edge_pass(hE, e_src, d_dst, z2d, z1d, False)
        hD = _tc_update(hD, T[0], T[1], cd0, cd1, None, We2d[k], Wd_self[k],
                        bd[k].reshape(1, _H), ln_d_g[k].reshape(1, _H),
                        ln_d_b[k].reshape(1, _H), False)

    stats = _tc_pool(hE, maskf)
    return _tc_head(stats, Wh1, bh1.reshape(1, _H), Wh2, bh2.reshape(1, 1))
